# Initial kernel scaffold; baseline (speedup 1.0000x reference)
import jax
import jax.numpy as jnp
from jax.experimental import pallas as pl


def kernel(x, edge_index, batch, W1, b1, W2, b2, fc1_W, fc1_b, fc2_W, fc2_b):
    raise NotImplementedError("write your pallas kernel here")


# speedup vs baseline: 41.2567x; 41.2567x over previous
import functools

import jax
import jax.numpy as jnp
from jax import lax
from jax.experimental import pallas as pl
from jax.experimental.pallas import tpu as pltpu
from jax.experimental.pallas import tpu_sc as plsc

N = 100000
E = 3200000
G = 64
F1 = 16

NC, NS = 2, 16
NW = NC * NS

LANE = 128
ROWS = 25088
EPAD = ROWS * LANE
PAD_E = EPAD - E
SB = 8

TRASH = 96
ACC_ROWS = N + TRASH
DEG_TILE = ACC_ROWS // NS
ZROWS = 272
ZITER = DEG_TILE // ZROWS

_MESH = plsc.VectorSubcoreMesh(core_axis_name="c", subcore_axis_name="s")


@functools.partial(
    pl.kernel,
    out_type=jax.ShapeDtypeStruct((NC * ACC_ROWS,), jnp.float32),
    mesh=_MESH,
    scratch_types=[
        pltpu.VMEM((SB, LANE), jnp.int32),
        pltpu.VMEM((LANE,), jnp.float32),
        pltpu.VMEM((DEG_TILE,), jnp.float32),
        pltpu.VMEM_SHARED((ACC_ROWS,), jnp.float32),
        pltpu.SemaphoreType.DMA,
    ],
)
def _deg_kernel(dst_hbm, out_hbm, idx_v, ones_v, zb_v, acc_sh, sem):
    core = lax.axis_index("c")
    tile = lax.axis_index("s")

    @pl.loop(0, LANE, step=16)
    def _(i):
        ones_v[pl.ds(i, 16)] = jnp.ones((16,), jnp.float32)

    @pl.loop(0, DEG_TILE, step=16)
    def _(i):
        zb_v[pl.ds(i, 16)] = jnp.zeros((16,), jnp.float32)

    pltpu.sync_copy(zb_v, acc_sh.at[pl.ds(tile * DEG_TILE, DEG_TILE)])
    plsc.subcore_barrier()

    w = core * NS + tile
    rows_per_w = ROWS // NW
    base = w * rows_per_w

    @pl.loop(0, rows_per_w, step=SB)
    def _(r):
        pltpu.sync_copy(dst_hbm.at[pl.ds(base + r, SB)], idx_v)
        cps = [
            pltpu.async_copy(ones_v, acc_sh.at[idx_v.at[b]], sem, add=True)
            for b in range(SB)
        ]
        for c in cps:
            c.wait()

    plsc.subcore_barrier()
    pltpu.sync_copy(acc_sh.at[pl.ds(tile * DEG_TILE, DEG_TILE)], zb_v)
    pltpu.sync_copy(
        zb_v,
        out_hbm.at[pl.ds(core * ACC_ROWS + tile * DEG_TILE, DEG_TILE)],
    )


@functools.partial(
    pl.kernel,
    out_type=jax.ShapeDtypeStruct((NC, ACC_ROWS, F1), jnp.float32),
    mesh=_MESH,
    scratch_types=[
        pltpu.VMEM((SB, LANE), jnp.int32),
        pltpu.VMEM((SB, LANE), jnp.int32),
        pltpu.VMEM((SB, LANE, F1), jnp.float32),
        pltpu.VMEM((ZROWS, F1), jnp.float32),
        pltpu.VMEM_SHARED((ACC_ROWS, F1), jnp.float32),
        pltpu.SemaphoreType.DMA,
        pltpu.SemaphoreType.DMA,
    ],
    compiler_params=pltpu.CompilerParams(use_tc_tiling_on_sc=False),
)
def _conv16_kernel(y_hbm, src_hbm, dst_hbm, out_hbm,
                   si_v, di_v, rows_v, zb_v, acc_sh, gsem, ssem):
    core = lax.axis_index("c")
    tile = lax.axis_index("s")

    @pl.loop(0, ZROWS)
    def _(i):
        zb_v[i, :] = jnp.zeros((16,), jnp.float32)

    @pl.loop(0, ZITER)
    def _(k):
        pltpu.sync_copy(
            zb_v, acc_sh.at[pl.ds(tile * DEG_TILE + k * ZROWS, ZROWS)]
        )
    plsc.subcore_barrier()

    w = core * NS + tile
    rows_per_w = ROWS // NW
    base = w * rows_per_w

    @pl.loop(0, rows_per_w, step=SB)
    def _(r):
        pltpu.sync_copy(src_hbm.at[pl.ds(base + r, SB)], si_v)
        pltpu.sync_copy(dst_hbm.at[pl.ds(base + r, SB)], di_v)
        gs = [
            pltpu.async_copy(y_hbm.at[si_v.at[b]], rows_v.at[b], gsem)
            for b in range(SB)
        ]
        for c in gs:
            c.wait()
        ss = [
            pltpu.async_copy(rows_v.at[b], acc_sh.at[di_v.at[b]], ssem, add=True)
            for b in range(SB)
        ]
        for c in ss:
            c.wait()

    plsc.subcore_barrier()

    @pl.loop(0, ZITER)
    def _(k):
        off = tile * DEG_TILE + k * ZROWS
        pltpu.sync_copy(acc_sh.at[pl.ds(off, ZROWS)], zb_v)
        pltpu.sync_copy(zb_v, out_hbm.at[core, pl.ds(off, ZROWS)])


_NB = 50
_BLK = N // _NB


def _tc_a(degA, degB, x, W1):
    def body(da, db, x_ref, w_ref, dinv_ref, y_ref):
        deg = da[...] + db[...] + 1.0
        dinv = lax.rsqrt(deg)
        dinv_ref[...] = dinv
        y_ref[...] = (
            jnp.dot(x_ref[...], w_ref[...], preferred_element_type=jnp.float32)
            * dinv
        )

    return pl.pallas_call(
        body,
        grid=(_NB,),
        in_specs=[
            pl.BlockSpec((_BLK, 1), lambda i: (i, 0)),
            pl.BlockSpec((_BLK, 1), lambda i: (i, 0)),
            pl.BlockSpec((_BLK, 2), lambda i: (i, 0)),
            pl.BlockSpec((2, F1), lambda i: (0, 0)),
        ],
        out_specs=[
            pl.BlockSpec((_BLK, 1), lambda i: (i, 0)),
            pl.BlockSpec((_BLK, F1), lambda i: (i, 0)),
        ],
        out_shape=[
            jax.ShapeDtypeStruct((N, 1), jnp.float32),
            jax.ShapeDtypeStruct((N, F1), jnp.float32),
        ],
    )(degA, degB, x, W1)


def _tc_b(accA, accB, y1, dinv, b1, W2a, W2b):
    def body(aa, ab, y_ref, di, b_ref, wa, wb, ya_ref, yb_ref):
        dinv = di[...]
        h = jnp.maximum(dinv * (aa[...] + ab[...] + y_ref[...]) + b_ref[...], 0.0)
        ya_ref[...] = (
            jnp.dot(h, wa[...], preferred_element_type=jnp.float32) * dinv
        )
        yb_ref[...] = (
            jnp.dot(h, wb[...], preferred_element_type=jnp.float32) * dinv
        )

    return pl.pallas_call(
        body,
        grid=(_NB,),
        in_specs=[
            pl.BlockSpec((_BLK, F1), lambda i: (i, 0)),
            pl.BlockSpec((_BLK, F1), lambda i: (i, 0)),
            pl.BlockSpec((_BLK, F1), lambda i: (i, 0)),
            pl.BlockSpec((_BLK, 1), lambda i: (i, 0)),
            pl.BlockSpec((1, F1), lambda i: (0, 0)),
            pl.BlockSpec((F1, F1), lambda i: (0, 0)),
            pl.BlockSpec((F1, F1), lambda i: (0, 0)),
        ],
        out_specs=[
            pl.BlockSpec((_BLK, F1), lambda i: (i, 0)),
            pl.BlockSpec((_BLK, F1), lambda i: (i, 0)),
        ],
        out_shape=[
            jax.ShapeDtypeStruct((N, F1), jnp.float32),
            jax.ShapeDtypeStruct((N, F1), jnp.float32),
        ],
    )(accA, accB, y1, dinv, b1, W2a, W2b)


def _tc_c(a2aA, a2aB, a2bA, a2bB, y2a, y2b, dinv, b2a, b2b, batch2d,
          fc1_Wa, fc1_Wb, fc1_b, fc2_W, fc2_b):
    def body(xaA, xaB, xbA, xbB, ya, yb, di, ba, bb, bt,
             wa, wb, w1b, w2, w2b, out_ref, sa, sb, cnt):
        pi = pl.program_id(0)

        @pl.when(pi == 0)
        def _():
            sa[...] = jnp.zeros_like(sa)
            sb[...] = jnp.zeros_like(sb)
            cnt[...] = jnp.zeros_like(cnt)

        dinv = di[...]
        ha = jnp.maximum(dinv * (xaA[...] + xaB[...] + ya[...]) + ba[...], 0.0)
        hb = jnp.maximum(dinv * (xbA[...] + xbB[...] + yb[...]) + bb[...], 0.0)
        seg = lax.broadcasted_iota(jnp.int32, (_BLK, G), 1)
        onehot = (bt[...] == seg).astype(jnp.float32)
        dn = (((0,), (0,)), ((), ()))
        sa[...] += lax.dot_general(onehot, ha, dn,
                                   preferred_element_type=jnp.float32)
        sb[...] += lax.dot_general(onehot, hb, dn,
                                   preferred_element_type=jnp.float32)
        cnt[...] += lax.dot_general(onehot, jnp.ones((_BLK, 1), jnp.float32),
                                    dn, preferred_element_type=jnp.float32)

        @pl.when(pi == _NB - 1)
        def _():
            c = jnp.maximum(cnt[...], 1.0)
            pa = sa[...] / c
            pb = sb[...] / c
            r = jnp.maximum(
                jnp.dot(pa, wa[...], preferred_element_type=jnp.float32)
                + jnp.dot(pb, wb[...], preferred_element_type=jnp.float32)
                + w1b[...],
                0.0,
            )
            out_ref[...] = (
                jnp.dot(r, w2[...], preferred_element_type=jnp.float32)
                + w2b[...]
            )

    full = lambda shape: pl.BlockSpec(shape, lambda i: (0, 0))
    blk = lambda w: pl.BlockSpec((_BLK, w), lambda i: (i, 0))
    return pl.pallas_call(
        body,
        grid=(_NB,),
        in_specs=[
            blk(F1), blk(F1), blk(F1), blk(F1), blk(F1), blk(F1), blk(1),
            full((1, F1)), full((1, F1)), blk(1),
            full((F1, F1)), full((F1, F1)), full((1, F1)),
            full((F1, 1)), full((1, 1)),
        ],
        out_specs=pl.BlockSpec((G, 1), lambda i: (0, 0)),
        out_shape=jax.ShapeDtypeStruct((G, 1), jnp.float32),
        scratch_shapes=[
            pltpu.VMEM((G, F1), jnp.float32),
            pltpu.VMEM((G, F1), jnp.float32),
            pltpu.VMEM((G, 1), jnp.float32),
        ],
    )(a2aA, a2aB, a2bA, a2bB, y2a, y2b, dinv, b2a, b2b, batch2d,
      fc1_Wa, fc1_Wb, fc1_b, fc2_W, fc2_b)


def kernel(x, edge_index, batch, W1, b1, W2, b2, fc1_W, fc1_b, fc2_W, fc2_b):
    ei = edge_index.astype(jnp.int32)
    pad = jnp.arange(PAD_E, dtype=jnp.int32)
    src_p = jnp.concatenate([ei[0], pad % N]).reshape(ROWS, LANE)
    dst_p = jnp.concatenate([ei[1], N + pad % TRASH]).reshape(ROWS, LANE)

    deg2 = _deg_kernel(dst_p).reshape(NC, ACC_ROWS)
    degA = deg2[0, :N].reshape(N, 1)
    degB = deg2[1, :N].reshape(N, 1)

    dinv, y1 = _tc_a(degA, degB, x, W1)

    acc1 = _conv16_kernel(y1, src_p, dst_p)
    y2a, y2b = _tc_b(acc1[0, :N], acc1[1, :N], y1, dinv,
                     b1.reshape(1, F1), W2[:, :F1], W2[:, F1:])

    acc2a = _conv16_kernel(y2a, src_p, dst_p)
    acc2b = _conv16_kernel(y2b, src_p, dst_p)

    out2d = _tc_c(
        acc2a[0, :N], acc2a[1, :N], acc2b[0, :N], acc2b[1, :N],
        y2a, y2b, dinv,
        b2[:F1].reshape(1, F1), b2[F1:].reshape(1, F1),
        batch.astype(jnp.int32).reshape(N, 1),
        fc1_W[:F1], fc1_W[F1:], fc1_b.reshape(1, F1),
        fc2_W, fc2_b.reshape(1, 1),
    )
    return out2d.reshape(G)


# speedup vs baseline: 46.7986x; 1.1343x over previous
import functools

import jax
import jax.numpy as jnp
from jax import lax
from jax.experimental import pallas as pl
from jax.experimental.pallas import tpu as pltpu
from jax.experimental.pallas import tpu_sc as plsc

N = 100000
E = 3200000
G = 64
F1 = 16

NC, NS = 2, 16
NW = NC * NS

LANE = 128
SB = 4
CHUNKS = 6272
EPAD = CHUNKS * SB * LANE
PAD_E = EPAD - E

TRASH = 96
ACC_ROWS = N + TRASH
TROWS = ACC_ROWS // NS
ZROWS = 136
ZITER = TROWS // ZROWS

_MESH = plsc.VectorSubcoreMesh(core_axis_name="c", subcore_axis_name="s")
_SC_PARAMS = pltpu.CompilerParams(use_tc_tiling_on_sc=False)


@functools.partial(
    pl.kernel,
    out_type=jax.ShapeDtypeStruct((NC * ACC_ROWS,), jnp.float32),
    mesh=_MESH,
    scratch_types=[
        pltpu.VMEM((2, SB, LANE), jnp.int32),
        pltpu.VMEM((LANE,), jnp.float32),
        pltpu.VMEM((TROWS,), jnp.float32),
        pltpu.VMEM_SHARED((ACC_ROWS,), jnp.float32),
        pltpu.SemaphoreType.DMA,
        pltpu.SemaphoreType.DMA,
    ],
)
def _deg_kernel(sidi_hbm, out_hbm, di_v, ones_v, zb_v, acc_sh, sem0, sem1):
    core = lax.axis_index("c")
    tile = lax.axis_index("s")
    sems = (sem0, sem1)

    @pl.loop(0, LANE, step=16)
    def _(i):
        ones_v[pl.ds(i, 16)] = jnp.ones((16,), jnp.float32)

    @pl.loop(0, TROWS, step=16)
    def _(i):
        zb_v[pl.ds(i, 16)] = jnp.zeros((16,), jnp.float32)

    pltpu.sync_copy(zb_v, acc_sh.at[pl.ds(tile * TROWS, TROWS)])
    plsc.subcore_barrier()

    w = core * NS + tile
    nchunks = CHUNKS // NW
    base = w * nchunks

    def fire(buf, c):
        pltpu.sync_copy(sidi_hbm.at[base + c, 2], di_v.at[buf])
        for b in range(SB):
            pltpu.async_copy(ones_v, acc_sh.at[di_v.at[buf, b]], sems[buf],
                             add=True)

    def drain(buf):
        for b in range(SB):
            pltpu.make_async_copy(ones_v, acc_sh.at[di_v.at[buf, b]],
                                  sems[buf]).wait()

    @pl.loop(0, nchunks // 2)
    def _(t):
        @pl.when(t > 0)
        def _():
            drain(0)
        fire(0, 2 * t)

        @pl.when(t > 0)
        def _():
            drain(1)
        fire(1, 2 * t + 1)

    drain(0)
    drain(1)
    plsc.subcore_barrier()
    pltpu.sync_copy(acc_sh.at[pl.ds(tile * TROWS, TROWS)], zb_v)
    pltpu.sync_copy(
        zb_v, out_hbm.at[pl.ds(core * ACC_ROWS + tile * TROWS, TROWS)]
    )


def _make_conv(split):
    yrows = 2 * N if split else N

    @functools.partial(
        pl.kernel,
        out_type=jax.ShapeDtypeStruct((NC, ACC_ROWS, F1), jnp.float32),
        mesh=_MESH,
        scratch_types=[
            pltpu.VMEM((2, 3, SB, LANE), jnp.int32),
            pltpu.VMEM((2, SB, LANE, F1), jnp.float32),
            pltpu.VMEM((ZROWS, F1), jnp.float32),
            pltpu.VMEM_SHARED((ACC_ROWS, F1), jnp.float32),
            pltpu.SemaphoreType.DMA,
            pltpu.SemaphoreType.DMA,
            pltpu.SemaphoreType.DMA,
            pltpu.SemaphoreType.DMA,
        ],
        compiler_params=_SC_PARAMS,
    )
    def conv(y_hbm, sidi_hbm, out_hbm, sidi_v, rows_v, zb_v, acc_sh,
             gs0, gs1, ss0, ss1):
        core = lax.axis_index("c")
        tile = lax.axis_index("s")
        gsems = (gs0, gs1)
        ssems = (ss0, ss1)

        @pl.loop(0, ZROWS)
        def _(i):
            zb_v[i, :] = jnp.zeros((16,), jnp.float32)

        @pl.loop(0, ZITER)
        def _(k):
            pltpu.sync_copy(
                zb_v, acc_sh.at[pl.ds(tile * TROWS + k * ZROWS, ZROWS)]
            )
        plsc.subcore_barrier()

        if split:
            nchunks = CHUNKS // NS
            base = tile * nchunks
            yrow = core
        else:
            nchunks = CHUNKS // NW
            base = (core * NS + tile) * nchunks
            yrow = 0

        def load_fire(buf, c):
            pltpu.sync_copy(sidi_hbm.at[base + c], sidi_v.at[buf])
            for b in range(SB):
                pltpu.async_copy(y_hbm.at[sidi_v.at[buf, yrow, b]],
                                 rows_v.at[buf, b], gsems[buf])

        def drain_g(buf):
            for b in range(SB):
                pltpu.make_async_copy(y_hbm.at[sidi_v.at[buf, yrow, b]],
                                      rows_v.at[buf, b], gsems[buf]).wait()

        def fire_s(buf):
            for b in range(SB):
                pltpu.async_copy(rows_v.at[buf, b],
                                 acc_sh.at[sidi_v.at[buf, 2, b]],
                                 ssems[buf], add=True)

        def drain_s(buf):
            for b in range(SB):
                pltpu.make_async_copy(rows_v.at[buf, b],
                                      acc_sh.at[sidi_v.at[buf, 2, b]],
                                      ssems[buf]).wait()

        @pl.loop(0, nchunks // 2)
        def _(t):
            @pl.when(t > 0)
            def _():
                drain_s(0)
            load_fire(0, 2 * t)

            @pl.when(t > 0)
            def _():
                drain_s(1)
            load_fire(1, 2 * t + 1)

            drain_g(0)
            fire_s(0)
            drain_g(1)
            fire_s(1)

        drain_s(0)
        drain_s(1)
        plsc.subcore_barrier()

        @pl.loop(0, ZITER)
        def _(k):
            off = tile * TROWS + k * ZROWS
            pltpu.sync_copy(acc_sh.at[pl.ds(off, ZROWS)], zb_v)
            pltpu.sync_copy(zb_v, out_hbm.at[core, pl.ds(off, ZROWS)])

    return conv


_conv_l1 = _make_conv(split=False)
_conv_l2 = _make_conv(split=True)


_NB = 50
_BLK = N // _NB


def _tc_a(degA, degB, x, W1):
    def body(da, db, x_ref, w_ref, dinv_ref, y_ref):
        deg = da[...] + db[...] + 1.0
        dinv = lax.rsqrt(deg)
        dinv_ref[...] = dinv
        y_ref[...] = (
            jnp.dot(x_ref[...], w_ref[...], preferred_element_type=jnp.float32)
            * dinv
        )

    return pl.pallas_call(
        body,
        grid=(_NB,),
        in_specs=[
            pl.BlockSpec((_BLK, 1), lambda i: (i, 0)),
            pl.BlockSpec((_BLK, 1), lambda i: (i, 0)),
            pl.BlockSpec((_BLK, 2), lambda i: (i, 0)),
            pl.BlockSpec((2, F1), lambda i: (0, 0)),
        ],
        out_specs=[
            pl.BlockSpec((_BLK, 1), lambda i: (i, 0)),
            pl.BlockSpec((_BLK, F1), lambda i: (i, 0)),
        ],
        out_shape=[
            jax.ShapeDtypeStruct((N, 1), jnp.float32),
            jax.ShapeDtypeStruct((N, F1), jnp.float32),
        ],
    )(degA, degB, x, W1)


def _tc_b(accA, accB, y1, dinv, b1, W2a, W2b):
    def body(aa, ab, y_ref, di, b_ref, wa, wb, ya_ref, yb_ref):
        dinv = di[...]
        h = jnp.maximum(dinv * (aa[...] + ab[...] + y_ref[...]) + b_ref[...], 0.0)
        ya_ref[...] = (
            jnp.dot(h, wa[...], preferred_element_type=jnp.float32) * dinv
        )
        yb_ref[...] = (
            jnp.dot(h, wb[...], preferred_element_type=jnp.float32) * dinv
        )

    return pl.pallas_call(
        body,
        grid=(_NB,),
        in_specs=[
            pl.BlockSpec((_BLK, F1), lambda i: (i, 0)),
            pl.BlockSpec((_BLK, F1), lambda i: (i, 0)),
            pl.BlockSpec((_BLK, F1), lambda i: (i, 0)),
            pl.BlockSpec((_BLK, 1), lambda i: (i, 0)),
            pl.BlockSpec((1, F1), lambda i: (0, 0)),
            pl.BlockSpec((F1, F1), lambda i: (0, 0)),
            pl.BlockSpec((F1, F1), lambda i: (0, 0)),
        ],
        out_specs=[
            pl.BlockSpec((_BLK, F1), lambda i: (i, 0)),
            pl.BlockSpec((_BLK, F1), lambda i: (i, 0)),
        ],
        out_shape=[
            jax.ShapeDtypeStruct((N, F1), jnp.float32),
            jax.ShapeDtypeStruct((N, F1), jnp.float32),
        ],
    )(accA, accB, y1, dinv, b1, W2a, W2b)


def _tc_c(a2a, a2b, y2a, y2b, dinv, b2a, b2b, batch2d,
          fc1_Wa, fc1_Wb, fc1_b, fc2_W, fc2_b):
    def body(xa, xb, ya, yb, di, ba, bb, bt,
             wa, wb, w1b, w2, w2b, out_ref, sa, sb, cnt):
        pi = pl.program_id(0)

        @pl.when(pi == 0)
        def _():
            sa[...] = jnp.zeros_like(sa)
            sb[...] = jnp.zeros_like(sb)
            cnt[...] = jnp.zeros_like(cnt)

        dinv = di[...]
        ha = jnp.maximum(dinv * (xa[...] + ya[...]) + ba[...], 0.0)
        hb = jnp.maximum(dinv * (xb[...] + yb[...]) + bb[...], 0.0)
        seg = lax.broadcasted_iota(jnp.int32, (_BLK, G), 1)
        onehot = (bt[...] == seg).astype(jnp.float32)
        dn = (((0,), (0,)), ((), ()))
        sa[...] += lax.dot_general(onehot, ha, dn,
                                   preferred_element_type=jnp.float32)
        sb[...] += lax.dot_general(onehot, hb, dn,
                                   preferred_element_type=jnp.float32)
        cnt[...] += lax.dot_general(onehot, jnp.ones((_BLK, 1), jnp.float32),
                                    dn, preferred_element_type=jnp.float32)

        @pl.when(pi == _NB - 1)
        def _():
            c = jnp.maximum(cnt[...], 1.0)
            pa = sa[...] / c
            pb = sb[...] / c
            r = jnp.maximum(
                jnp.dot(pa, wa[...], preferred_element_type=jnp.float32)
                + jnp.dot(pb, wb[...], preferred_element_type=jnp.float32)
                + w1b[...],
                0.0,
            )
            out_ref[...] = (
                jnp.dot(r, w2[...], preferred_element_type=jnp.float32)
                + w2b[...]
            )

    full = lambda shape: pl.BlockSpec(shape, lambda i: (0, 0))
    blk = lambda w: pl.BlockSpec((_BLK, w), lambda i: (i, 0))
    return pl.pallas_call(
        body,
        grid=(_NB,),
        in_specs=[
            blk(F1), blk(F1), blk(F1), blk(F1), blk(1),
            full((1, F1)), full((1, F1)), blk(1),
            full((F1, F1)), full((F1, F1)), full((1, F1)),
            full((F1, 1)), full((1, 1)),
        ],
        out_specs=pl.BlockSpec((G, 1), lambda i: (0, 0)),
        out_shape=jax.ShapeDtypeStruct((G, 1), jnp.float32),
        scratch_shapes=[
            pltpu.VMEM((G, F1), jnp.float32),
            pltpu.VMEM((G, F1), jnp.float32),
            pltpu.VMEM((G, 1), jnp.float32),
        ],
    )(a2a, a2b, y2a, y2b, dinv, b2a, b2b, batch2d,
      fc1_Wa, fc1_Wb, fc1_b, fc2_W, fc2_b)


def kernel(x, edge_index, batch, W1, b1, W2, b2, fc1_W, fc1_b, fc2_W, fc2_b):
    ei = edge_index.astype(jnp.int32)
    pad = jnp.arange(PAD_E, dtype=jnp.int32)
    srcr = jnp.concatenate([ei[0], pad % N]).reshape(CHUNKS, SB, LANE)
    dstr = jnp.concatenate([ei[1], N + pad % TRASH]).reshape(CHUNKS, SB, LANE)
    sidi = jnp.stack([srcr, srcr + N, dstr], axis=1)

    deg2 = _deg_kernel(sidi).reshape(NC, ACC_ROWS)
    degA = deg2[0, :N].reshape(N, 1)
    degB = deg2[1, :N].reshape(N, 1)

    dinv, y1 = _tc_a(degA, degB, x, W1)

    acc1 = _conv_l1(y1, sidi)
    y2a, y2b = _tc_b(acc1[0, :N], acc1[1, :N], y1, dinv,
                     b1.reshape(1, F1), W2[:, :F1], W2[:, F1:])

    acc2 = _conv_l2(jnp.concatenate([y2a, y2b]), sidi)

    out2d = _tc_c(
        acc2[0, :N], acc2[1, :N],
        y2a, y2b, dinv,
        b2[:F1].reshape(1, F1), b2[F1:].reshape(1, F1),
        batch.astype(jnp.int32).reshape(N, 1),
        fc1_W[:F1], fc1_W[F1:], fc1_b.reshape(1, F1),
        fc2_W, fc2_b.reshape(1, 1),
    )
    return out2d.reshape(G)


# speedup vs baseline: 52.3930x; 1.1195x over previous
import functools

import jax
import jax.numpy as jnp
from jax import lax
from jax.experimental import pallas as pl
from jax.experimental.pallas import tpu as pltpu
from jax.experimental.pallas import tpu_sc as plsc

N = 100000
E = 3200000
G = 64
F1 = 16

NC, NS = 2, 16
NW = NC * NS

LANE = 128
SB = 4
CHUNKS = 6272
EPAD = CHUNKS * SB * LANE
PAD_E = EPAD - E

TRASH = 96
ACC_ROWS = N + TRASH
TROWS = ACC_ROWS // NS
ZROWS = 136
ZITER = TROWS // ZROWS

_MESH = plsc.VectorSubcoreMesh(core_axis_name="c", subcore_axis_name="s")
_SC_PARAMS = pltpu.CompilerParams(use_tc_tiling_on_sc=False)


@functools.partial(
    pl.kernel,
    out_type=jax.ShapeDtypeStruct((NC * ACC_ROWS,), jnp.float32),
    mesh=_MESH,
    scratch_types=[
        pltpu.VMEM((2, SB, LANE), jnp.int32),
        pltpu.VMEM((LANE,), jnp.float32),
        pltpu.VMEM((TROWS,), jnp.float32),
        pltpu.VMEM_SHARED((ACC_ROWS,), jnp.float32),
        pltpu.SemaphoreType.DMA,
        pltpu.SemaphoreType.DMA,
    ],
    compiler_params=pltpu.CompilerParams(use_tc_tiling_on_sc=False),
)
def _deg_kernel(sidi_hbm, out_hbm, di_v, ones_v, zb_v, acc_sh, sem0, sem1):
    core = lax.axis_index("c")
    tile = lax.axis_index("s")
    sems = (sem0, sem1)

    @pl.loop(0, LANE, step=16)
    def _(i):
        ones_v[pl.ds(i, 16)] = jnp.ones((16,), jnp.float32)

    @pl.loop(0, TROWS, step=16)
    def _(i):
        zb_v[pl.ds(i, 16)] = jnp.zeros((16,), jnp.float32)

    pltpu.sync_copy(zb_v, acc_sh.at[pl.ds(tile * TROWS, TROWS)])
    plsc.subcore_barrier()

    w = core * NS + tile
    nchunks = CHUNKS // NW
    base = w * nchunks

    def fire(buf, c):
        pltpu.sync_copy(sidi_hbm.at[base + c, 2], di_v.at[buf])
        for b in range(SB):
            pltpu.async_copy(ones_v, acc_sh.at[di_v.at[buf, b]], sems[buf],
                             add=True)

    def drain(buf):
        for b in range(SB):
            pltpu.make_async_copy(ones_v, acc_sh.at[di_v.at[buf, b]],
                                  sems[buf]).wait()

    @pl.loop(0, nchunks // 2)
    def _(t):
        @pl.when(t > 0)
        def _():
            drain(0)
        fire(0, 2 * t)

        @pl.when(t > 0)
        def _():
            drain(1)
        fire(1, 2 * t + 1)

    drain(0)
    drain(1)
    plsc.subcore_barrier()
    pltpu.sync_copy(acc_sh.at[pl.ds(tile * TROWS, TROWS)], zb_v)
    pltpu.sync_copy(
        zb_v, out_hbm.at[pl.ds(core * ACC_ROWS + tile * TROWS, TROWS)]
    )


def _make_conv(split):
    @functools.partial(
        pl.kernel,
        out_type=jax.ShapeDtypeStruct((NC, ACC_ROWS, F1), jnp.float32),
        mesh=_MESH,
        scratch_types=[
            pltpu.VMEM((2, 3, SB, LANE), jnp.int32),
            pltpu.VMEM((2, SB, LANE, F1), jnp.float32),
            pltpu.VMEM((ZROWS, F1), jnp.float32),
            pltpu.VMEM_SHARED((ACC_ROWS, F1), jnp.float32),
            pltpu.SemaphoreType.DMA,
            pltpu.SemaphoreType.DMA,
            pltpu.SemaphoreType.DMA,
            pltpu.SemaphoreType.DMA,
        ],
        compiler_params=_SC_PARAMS,
    )
    def conv(y_hbm, sidi_hbm, out_hbm, sidi_v, rows_v, zb_v, acc_sh,
             gs0, gs1, ss0, ss1):
        core = lax.axis_index("c")
        tile = lax.axis_index("s")
        gsems = (gs0, gs1)
        ssems = (ss0, ss1)

        @pl.loop(0, ZROWS)
        def _(i):
            zb_v[i, :] = jnp.zeros((16,), jnp.float32)

        @pl.loop(0, ZITER)
        def _(k):
            pltpu.sync_copy(
                zb_v, acc_sh.at[pl.ds(tile * TROWS + k * ZROWS, ZROWS)]
            )
        plsc.subcore_barrier()

        if split:
            nchunks = CHUNKS // NS
            base = tile * nchunks
            yrow = core
        else:
            nchunks = CHUNKS // NW
            base = (core * NS + tile) * nchunks
            yrow = 0

        def load_fire(buf, c):
            pltpu.sync_copy(sidi_hbm.at[base + c], sidi_v.at[buf])
            for b in range(SB):
                pltpu.async_copy(y_hbm.at[sidi_v.at[buf, yrow, b]],
                                 rows_v.at[buf, b], gsems[buf])

        def drain_g(buf):
            for b in range(SB):
                pltpu.make_async_copy(y_hbm.at[sidi_v.at[buf, yrow, b]],
                                      rows_v.at[buf, b], gsems[buf]).wait()

        def fire_s(buf):
            for b in range(SB):
                pltpu.async_copy(rows_v.at[buf, b],
                                 acc_sh.at[sidi_v.at[buf, 2, b]],
                                 ssems[buf], add=True)

        def drain_s(buf):
            for b in range(SB):
                pltpu.make_async_copy(rows_v.at[buf, b],
                                      acc_sh.at[sidi_v.at[buf, 2, b]],
                                      ssems[buf]).wait()

        @pl.loop(0, nchunks // 2)
        def _(t):
            @pl.when(t > 0)
            def _():
                drain_s(0)
            load_fire(0, 2 * t)

            @pl.when(t > 0)
            def _():
                drain_s(1)
            load_fire(1, 2 * t + 1)

            drain_g(0)
            fire_s(0)
            drain_g(1)
            fire_s(1)

        drain_s(0)
        drain_s(1)
        plsc.subcore_barrier()

        @pl.loop(0, ZITER)
        def _(k):
            off = tile * TROWS + k * ZROWS
            pltpu.sync_copy(acc_sh.at[pl.ds(off, ZROWS)], zb_v)
            pltpu.sync_copy(zb_v, out_hbm.at[core, pl.ds(off, ZROWS)])

    return conv


_conv_l1 = _make_conv(split=False)
_conv_l2 = _make_conv(split=True)


_NB = 32
_BLK = ACC_ROWS // _NB


def _tc_a(deg2, xp, W1):
    def body(da, db, x_ref, w_ref, dinv_ref, y_ref):
        deg = da[...] + db[...] + 1.0
        dinv = lax.rsqrt(deg)
        dinv_ref[...] = dinv
        y_ref[...] = (
            jnp.dot(x_ref[...], w_ref[...], preferred_element_type=jnp.float32)
            * dinv
        )

    return pl.pallas_call(
        body,
        grid=(_NB,),
        in_specs=[
            pl.BlockSpec((_BLK, 1), lambda i: (i, 0)),
            pl.BlockSpec((_BLK, 1), lambda i: (i + _NB, 0)),
            pl.BlockSpec((_BLK, 2), lambda i: (i, 0)),
            pl.BlockSpec((2, F1), lambda i: (0, 0)),
        ],
        out_specs=[
            pl.BlockSpec((_BLK, 1), lambda i: (i, 0)),
            pl.BlockSpec((_BLK, F1), lambda i: (i, 0)),
        ],
        out_shape=[
            jax.ShapeDtypeStruct((ACC_ROWS, 1), jnp.float32),
            jax.ShapeDtypeStruct((ACC_ROWS, F1), jnp.float32),
        ],
    )(deg2, deg2, xp, W1)


def _tc_b(acc1, y1, dinv, b1, W2a, W2b):
    def body(acc, y_ref, di, b_ref, wa, wb, out_ref):
        dinv = di[...]
        h = jnp.maximum(
            dinv * (acc[0] + acc[1] + y_ref[...]) + b_ref[...], 0.0
        )
        out_ref[0] = (
            jnp.dot(h, wa[...], preferred_element_type=jnp.float32) * dinv
        )
        out_ref[1] = (
            jnp.dot(h, wb[...], preferred_element_type=jnp.float32) * dinv
        )

    return pl.pallas_call(
        body,
        grid=(_NB,),
        in_specs=[
            pl.BlockSpec((2, _BLK, F1), lambda i: (0, i, 0)),
            pl.BlockSpec((_BLK, F1), lambda i: (i, 0)),
            pl.BlockSpec((_BLK, 1), lambda i: (i, 0)),
            pl.BlockSpec((1, F1), lambda i: (0, 0)),
            pl.BlockSpec((F1, F1), lambda i: (0, 0)),
            pl.BlockSpec((F1, F1), lambda i: (0, 0)),
        ],
        out_specs=pl.BlockSpec((2, _BLK, F1), lambda i: (0, i, 0)),
        out_shape=jax.ShapeDtypeStruct((2, ACC_ROWS, F1), jnp.float32),
    )(acc1, y1, dinv, b1, W2a, W2b)


def _tc_c(acc2, y2, dinv, b2a, b2b, batchp,
          fc1_Wa, fc1_Wb, fc1_b, fc2_W, fc2_b):
    def body(xab, yab, di, ba, bb, bt,
             wa, wb, w1b, w2, w2b, out_ref, sa, sb, cnt):
        pi = pl.program_id(0)

        @pl.when(pi == 0)
        def _():
            sa[...] = jnp.zeros_like(sa)
            sb[...] = jnp.zeros_like(sb)
            cnt[...] = jnp.zeros_like(cnt)

        dinv = di[...]
        ha = jnp.maximum(dinv * (xab[0] + yab[0]) + ba[...], 0.0)
        hb = jnp.maximum(dinv * (xab[1] + yab[1]) + bb[...], 0.0)
        seg = lax.broadcasted_iota(jnp.int32, (_BLK, G), 1)
        onehot = (bt[...] == seg).astype(jnp.float32)
        dn = (((0,), (0,)), ((), ()))
        sa[...] += lax.dot_general(onehot, ha, dn,
                                   preferred_element_type=jnp.float32)
        sb[...] += lax.dot_general(onehot, hb, dn,
                                   preferred_element_type=jnp.float32)
        cnt[...] += lax.dot_general(onehot, jnp.ones((_BLK, 1), jnp.float32),
                                    dn, preferred_element_type=jnp.float32)

        @pl.when(pi == _NB - 1)
        def _():
            c = jnp.maximum(cnt[...], 1.0)
            pa = sa[...] / c
            pb = sb[...] / c
            r = jnp.maximum(
                jnp.dot(pa, wa[...], preferred_element_type=jnp.float32)
                + jnp.dot(pb, wb[...], preferred_element_type=jnp.float32)
                + w1b[...],
                0.0,
            )
            out_ref[...] = (
                jnp.dot(r, w2[...], preferred_element_type=jnp.float32)
                + w2b[...]
            )

    full = lambda shape: pl.BlockSpec(shape, lambda i: (0, 0))
    blk3 = pl.BlockSpec((2, _BLK, F1), lambda i: (0, i, 0))
    blk = lambda w: pl.BlockSpec((_BLK, w), lambda i: (i, 0))
    return pl.pallas_call(
        body,
        grid=(_NB,),
        in_specs=[
            blk3, blk3, blk(1),
            full((1, F1)), full((1, F1)), blk(1),
            full((F1, F1)), full((F1, F1)), full((1, F1)),
            full((F1, 1)), full((1, 1)),
        ],
        out_specs=pl.BlockSpec((G, 1), lambda i: (0, 0)),
        out_shape=jax.ShapeDtypeStruct((G, 1), jnp.float32),
        scratch_shapes=[
            pltpu.VMEM((G, F1), jnp.float32),
            pltpu.VMEM((G, F1), jnp.float32),
            pltpu.VMEM((G, 1), jnp.float32),
        ],
    )(acc2, y2, dinv, b2a, b2b, batchp,
      fc1_Wa, fc1_Wb, fc1_b, fc2_W, fc2_b)


def kernel(x, edge_index, batch, W1, b1, W2, b2, fc1_W, fc1_b, fc2_W, fc2_b):
    ei = edge_index.astype(jnp.int32)
    pad = jnp.arange(PAD_E, dtype=jnp.int32)
    srcr = jnp.concatenate([ei[0], pad % N]).reshape(CHUNKS, SB, LANE)
    dstr = jnp.concatenate([ei[1], N + pad % TRASH]).reshape(CHUNKS, SB, LANE)
    sidi = jnp.stack([srcr, srcr + ACC_ROWS, dstr], axis=1)

    xp = jnp.concatenate([x, jnp.zeros((TRASH, 2), jnp.float32)])
    batchp = jnp.concatenate(
        [batch.astype(jnp.int32), jnp.full((TRASH,), 127, jnp.int32)]
    ).reshape(ACC_ROWS, 1)

    deg2 = _deg_kernel(sidi).reshape(NC * ACC_ROWS, 1)

    dinv, y1 = _tc_a(deg2, xp, W1)

    acc1 = _conv_l1(y1, sidi)
    y2 = _tc_b(acc1, y1, dinv, b1.reshape(1, F1), W2[:, :F1], W2[:, F1:])

    acc2 = _conv_l2(y2.reshape(2 * ACC_ROWS, F1), sidi)

    out2d = _tc_c(
        acc2, y2, dinv,
        b2[:F1].reshape(1, F1), b2[F1:].reshape(1, F1),
        batchp,
        fc1_W[:F1], fc1_W[F1:], fc1_b.reshape(1, F1),
        fc2_W, fc2_b.reshape(1, 1),
    )
    return out2d.reshape(G)


# speedup vs baseline: 63.2666x; 1.2075x over previous
import functools

import jax
import jax.numpy as jnp
from jax import lax
from jax.experimental import pallas as pl
from jax.experimental.pallas import tpu as pltpu
from jax.experimental.pallas import tpu_sc as plsc

N = 100000
E = 3200000
G = 64
F1 = 16

NC, NS = 2, 16
NW = NC * NS

LANE = 128
SB = 4
CHUNKS = 6272
EPAD = CHUNKS * SB * LANE
PAD_E = EPAD - E

TRASH = 96
A = N + TRASH
TROWS = A // NS
ZROWS = 136
ZITER = TROWS // ZROWS

VROWS = A * F1 // LANE
DROWS = A // LANE

_MESH = plsc.VectorSubcoreMesh(core_axis_name="c", subcore_axis_name="s")
_SC_PARAMS = pltpu.CompilerParams(use_tc_tiling_on_sc=False)


@functools.partial(
    pl.kernel,
    out_type=jax.ShapeDtypeStruct((NC * A,), jnp.float32),
    mesh=_MESH,
    scratch_types=[
        pltpu.VMEM((2, SB, LANE), jnp.int32),
        pltpu.VMEM((LANE,), jnp.float32),
        pltpu.VMEM((TROWS,), jnp.float32),
        pltpu.VMEM_SHARED((A,), jnp.float32),
        pltpu.SemaphoreType.DMA,
        pltpu.SemaphoreType.DMA,
    ],
    compiler_params=_SC_PARAMS,
)
def _deg_kernel(dst_hbm, out_hbm, di_v, ones_v, zb_v, acc_sh, sem0, sem1):
    core = lax.axis_index("c")
    tile = lax.axis_index("s")
    sems = (sem0, sem1)

    @pl.loop(0, LANE, step=16)
    def _(i):
        ones_v[pl.ds(i, 16)] = jnp.ones((16,), jnp.float32)

    @pl.loop(0, TROWS, step=16)
    def _(i):
        zb_v[pl.ds(i, 16)] = jnp.zeros((16,), jnp.float32)

    pltpu.sync_copy(zb_v, acc_sh.at[pl.ds(tile * TROWS, TROWS)])
    plsc.subcore_barrier()

    w = core * NS + tile
    nchunks = CHUNKS // NW
    base = w * nchunks

    def fire(buf, c):
        pltpu.sync_copy(dst_hbm.at[base + c], di_v.at[buf])
        for b in range(SB):
            pltpu.async_copy(ones_v, acc_sh.at[di_v.at[buf, b]], sems[buf],
                             add=True)

    def drain(buf):
        for b in range(SB):
            pltpu.make_async_copy(ones_v, acc_sh.at[di_v.at[buf, b]],
                                  sems[buf]).wait()

    @pl.loop(0, nchunks // 2)
    def _(t):
        @pl.when(t > 0)
        def _():
            drain(0)
        fire(0, 2 * t)

        @pl.when(t > 0)
        def _():
            drain(1)
        fire(1, 2 * t + 1)

    drain(0)
    drain(1)
    plsc.subcore_barrier()
    pltpu.sync_copy(acc_sh.at[pl.ds(tile * TROWS, TROWS)], zb_v)
    pltpu.sync_copy(zb_v, out_hbm.at[pl.ds(core * A + tile * TROWS, TROWS)])


def _make_conv(split):
    @functools.partial(
        pl.kernel,
        out_type=jax.ShapeDtypeStruct((NC, A, F1), jnp.float32),
        mesh=_MESH,
        scratch_types=[
            pltpu.VMEM((2, SB, LANE), jnp.int32),
            pltpu.VMEM((2, SB, LANE), jnp.int32),
            pltpu.VMEM((2, SB, LANE, F1), jnp.float32),
            pltpu.VMEM((ZROWS, F1), jnp.float32),
            pltpu.VMEM_SHARED((A, F1), jnp.float32),
            pltpu.SemaphoreType.DMA,
            pltpu.SemaphoreType.DMA,
            pltpu.SemaphoreType.DMA,
            pltpu.SemaphoreType.DMA,
        ],
        compiler_params=_SC_PARAMS,
    )
    def conv(y_hbm, src2_hbm, dst_hbm, out_hbm, si_v, di_v, rows_v, zb_v,
             acc_sh, gs0, gs1, ss0, ss1):
        core = lax.axis_index("c")
        tile = lax.axis_index("s")
        gsems = (gs0, gs1)
        ssems = (ss0, ss1)

        @pl.loop(0, ZROWS)
        def _(i):
            zb_v[i, :] = jnp.zeros((16,), jnp.float32)

        @pl.loop(0, ZITER)
        def _(k):
            pltpu.sync_copy(
                zb_v, acc_sh.at[pl.ds(tile * TROWS + k * ZROWS, ZROWS)]
            )
        plsc.subcore_barrier()

        if split:
            nchunks = CHUNKS // NS
            base = tile * nchunks
            yplane = core
        else:
            nchunks = CHUNKS // NW
            base = (core * NS + tile) * nchunks
            yplane = 0

        def load_fire(buf, c):
            pltpu.sync_copy(src2_hbm.at[yplane, base + c], si_v.at[buf])
            pltpu.sync_copy(dst_hbm.at[base + c], di_v.at[buf])
            for b in range(SB):
                pltpu.async_copy(y_hbm.at[si_v.at[buf, b]],
                                 rows_v.at[buf, b], gsems[buf])

        def drain_g(buf):
            for b in range(SB):
                pltpu.make_async_copy(y_hbm.at[si_v.at[buf, b]],
                                      rows_v.at[buf, b], gsems[buf]).wait()

        def fire_s(buf):
            for b in range(SB):
                pltpu.async_copy(rows_v.at[buf, b],
                                 acc_sh.at[di_v.at[buf, b]],
                                 ssems[buf], add=True)

        def drain_s(buf):
            for b in range(SB):
                pltpu.make_async_copy(rows_v.at[buf, b],
                                      acc_sh.at[di_v.at[buf, b]],
                                      ssems[buf]).wait()

        @pl.loop(0, nchunks // 2)
        def _(t):
            @pl.when(t > 0)
            def _():
                drain_s(0)
            load_fire(0, 2 * t)

            @pl.when(t > 0)
            def _():
                drain_s(1)
            load_fire(1, 2 * t + 1)

            drain_g(0)
            fire_s(0)
            drain_g(1)
            fire_s(1)

        drain_s(0)
        drain_s(1)
        plsc.subcore_barrier()

        @pl.loop(0, ZITER)
        def _(k):
            off = tile * TROWS + k * ZROWS
            pltpu.sync_copy(acc_sh.at[pl.ds(off, ZROWS)], zb_v)
            pltpu.sync_copy(zb_v, out_hbm.at[core, pl.ds(off, ZROWS)])

    return conv


_conv_l1 = _make_conv(split=False)
_conv_l2 = _make_conv(split=True)


_NBV = 4
_BRV = VROWS // _NBV


def _tc_dinv(deg2v):
    def body(d_ref, o_ref):
        o_ref[...] = lax.rsqrt(d_ref[0] + d_ref[1] + 1.0)

    return pl.pallas_call(
        body,
        grid=(1,),
        in_specs=[pl.BlockSpec((2, DROWS, LANE), lambda i: (0, 0, 0))],
        out_specs=pl.BlockSpec((DROWS, LANE), lambda i: (0, 0)),
        out_shape=jax.ShapeDtypeStruct((DROWS, LANE), jnp.float32),
    )(deg2v)


def _tc_y1(x0e, x1e, dinve, w0t, w1t):
    def body(x0, x1, de, w0, w1, o_ref):
        o_ref[...] = (x0[...] * w0[...] + x1[...] * w1[...]) * de[...]

    blk = pl.BlockSpec((_BRV, LANE), lambda i: (i, 0))
    return pl.pallas_call(
        body,
        grid=(_NBV,),
        in_specs=[blk, blk, blk,
                  pl.BlockSpec((1, LANE), lambda i: (0, 0)),
                  pl.BlockSpec((1, LANE), lambda i: (0, 0))],
        out_specs=blk,
        out_shape=jax.ShapeDtypeStruct((VROWS, LANE), jnp.float32),
    )(x0e, x1e, dinve, w0t, w1t)


def _tc_b(acc1v, y1v, dinve, b1t, Ma, Mb):
    def body(acc, y_ref, de, bt, ma, mb, o_ref):
        dinv = de[...]
        h = jnp.maximum(dinv * (acc[0] + acc[1] + y_ref[...]) + bt[...], 0.0)
        o_ref[0] = (
            jnp.dot(h, ma[...], preferred_element_type=jnp.float32) * dinv
        )
        o_ref[1] = (
            jnp.dot(h, mb[...], preferred_element_type=jnp.float32) * dinv
        )

    blk = pl.BlockSpec((_BRV, LANE), lambda i: (i, 0))
    blk3 = pl.BlockSpec((2, _BRV, LANE), lambda i: (0, i, 0))
    full = lambda s: pl.BlockSpec(s, lambda i: (0, 0))
    return pl.pallas_call(
        body,
        grid=(_NBV,),
        in_specs=[blk3, blk, blk, full((1, LANE)),
                  full((LANE, LANE)), full((LANE, LANE))],
        out_specs=blk3,
        out_shape=jax.ShapeDtypeStruct((2, VROWS, LANE), jnp.float32),
    )(acc1v, y1v, dinve, b1t, Ma, Mb)


def _tc_head(acc2v, y2v, dinve, b2at, b2bt, batchv,
             fc1_Wa, fc1_Wb, fc1_b, fc2_W, fc2_b):
    def body(acc, yab, de, ba, bb, bt, wa, wb, w1b, w2, w2b,
             out_ref, sa, sb, cnt):
        pi = pl.program_id(0)

        @pl.when(pi == 0)
        def _():
            sa[...] = jnp.zeros_like(sa)
            sb[...] = jnp.zeros_like(sb)
            cnt[...] = jnp.zeros_like(cnt)

        dinv = de[...]
        ha = jnp.maximum(dinv * (acc[0] + yab[0]) + ba[...], 0.0)
        hb = jnp.maximum(dinv * (acc[1] + yab[1]) + bb[...], 0.0)
        seg = lax.broadcasted_iota(jnp.int32, (_BRV, G), 1)
        bt_all = bt[...]
        dn = (((0,), (0,)), ((), ()))
        ones = jnp.ones((_BRV, 1), jnp.float32)
        for p in range(8):
            onehot = (bt_all[:, p:p + 1] == seg).astype(jnp.float32)
            sa[...] += lax.dot_general(onehot, ha[:, 16 * p:16 * (p + 1)], dn,
                                       preferred_element_type=jnp.float32)
            sb[...] += lax.dot_general(onehot, hb[:, 16 * p:16 * (p + 1)], dn,
                                       preferred_element_type=jnp.float32)
            cnt[...] += lax.dot_general(onehot, ones, dn,
                                        preferred_element_type=jnp.float32)

        @pl.when(pi == _NBV - 1)
        def _():
            c = jnp.maximum(cnt[...], 1.0)
            pa = sa[...] / c
            pb = sb[...] / c
            r = jnp.maximum(
                jnp.dot(pa, wa[...], preferred_element_type=jnp.float32)
                + jnp.dot(pb, wb[...], preferred_element_type=jnp.float32)
                + w1b[...],
                0.0,
            )
            out_ref[...] = (
                jnp.dot(r, w2[...], preferred_element_type=jnp.float32)
                + w2b[...]
            )

    blk = pl.BlockSpec((_BRV, LANE), lambda i: (i, 0))
    blk3 = pl.BlockSpec((2, _BRV, LANE), lambda i: (0, i, 0))
    blk8 = pl.BlockSpec((_BRV, 8), lambda i: (i, 0))
    full = lambda s: pl.BlockSpec(s, lambda i: (0, 0))
    return pl.pallas_call(
        body,
        grid=(_NBV,),
        in_specs=[blk3, blk3, blk, full((1, LANE)), full((1, LANE)), blk8,
                  full((F1, F1)), full((F1, F1)), full((1, F1)),
                  full((F1, 1)), full((1, 1))],
        out_specs=pl.BlockSpec((G, 1), lambda i: (0, 0)),
        out_shape=jax.ShapeDtypeStruct((G, 1), jnp.float32),
        scratch_shapes=[
            pltpu.VMEM((G, F1), jnp.float32),
            pltpu.VMEM((G, F1), jnp.float32),
            pltpu.VMEM((G, 1), jnp.float32),
        ],
    )(acc2v, y2v, dinve, b2at, b2bt, batchv,
      fc1_Wa, fc1_Wb, fc1_b, fc2_W, fc2_b)


def kernel(x, edge_index, batch, W1, b1, W2, b2, fc1_W, fc1_b, fc2_W, fc2_b):
    ei = edge_index.astype(jnp.int32)
    pad = jnp.arange(PAD_E, dtype=jnp.int32)
    dstp = jnp.concatenate([ei[1], N + pad % TRASH]).reshape(CHUNKS, SB, LANE)
    src_all = jnp.concatenate([ei[0], pad % N])
    srcp2 = jnp.stack([src_all, src_all + A]).reshape(2, CHUNKS, SB, LANE)

    xp = jnp.concatenate([x, jnp.zeros((TRASH, 2), jnp.float32)])
    x0e = jnp.broadcast_to(xp[:, 0:1], (A, F1)).reshape(VROWS, LANE)
    x1e = jnp.broadcast_to(xp[:, 1:2], (A, F1)).reshape(VROWS, LANE)
    batchv = jnp.concatenate(
        [batch.astype(jnp.int32), jnp.full((TRASH,), 127, jnp.int32)]
    ).reshape(VROWS, 8)

    w0t = jnp.tile(W1[0], 8).reshape(1, LANE)
    w1t = jnp.tile(W1[1], 8).reshape(1, LANE)
    b1t = jnp.tile(b1, 8).reshape(1, LANE)
    b2at = jnp.tile(b2[:F1], 8).reshape(1, LANE)
    b2bt = jnp.tile(b2[F1:], 8).reshape(1, LANE)
    eye8 = jnp.eye(8, dtype=jnp.float32)
    Ma = jnp.kron(eye8, W2[:, :F1])
    Mb = jnp.kron(eye8, W2[:, F1:])

    deg2 = _deg_kernel(dstp)
    dinvv = _tc_dinv(deg2.reshape(2, DROWS, LANE))
    dinve = jnp.broadcast_to(
        dinvv.reshape(A, 1), (A, F1)
    ).reshape(VROWS, LANE)

    y1v = _tc_y1(x0e, x1e, dinve, w0t, w1t)

    acc1 = _conv_l1(y1v.reshape(A, F1), srcp2, dstp)
    y2v = _tc_b(acc1.reshape(2, VROWS, LANE), y1v, dinve, b1t, Ma, Mb)

    acc2 = _conv_l2(y2v.reshape(2 * A, F1), srcp2, dstp)

    out2d = _tc_head(
        acc2.reshape(2, VROWS, LANE), y2v, dinve, b2at, b2bt, batchv,
        fc1_W[:F1], fc1_W[F1:], fc1_b.reshape(1, F1),
        fc2_W, fc2_b.reshape(1, 1),
    )
    return out2d.reshape(G)
